# baseline (speedup 1.0000x reference)
import jax
import jax.numpy as jnp
from jax import lax
from jax.experimental import pallas as pl
from jax.experimental.pallas import tpu as pltpu
from jax.experimental.pallas import tpu_sc as plsc

B, L, D = 4, 4096, 1024
NTOK = B * L
LANES = 16
NC, NS = 2, 16
NW = NC * NS
TPW = NTOK // NW
CH = 32
NCHUNK = TPW // CH
NBUF = 3
JB = 16
NJB = (D // LANES) // JB


def _sc_body(x_hbm, tid_hbm, emb_hbm, out_hbm,
             emb_v, tid_v, buf0, buf1, buf2,
             si0, si1, si2, so0, so1, so2):
    c = lax.axis_index("c")
    s = lax.axis_index("s")
    wid = s * NC + c
    base = wid * TPW

    emb_cp = pltpu.async_copy(emb_hbm, emb_v, so0)
    tid_cp = pltpu.async_copy(tid_hbm.at[pl.ds(base, TPW)], tid_v, so1)

    bufs = (buf0, buf1, buf2)
    isems = (si0, si1, si2)
    osems = (so0, so1, so2)
    in_copies = [None] * NCHUNK
    out_copies = [None] * NCHUNK

    def start_in(ch):
        in_copies[ch] = pltpu.async_copy(
            x_hbm.at[pl.ds(base + ch * CH, CH)], bufs[ch % NBUF],
            isems[ch % NBUF])

    def start_out(ch):
        out_copies[ch] = pltpu.async_copy(
            bufs[ch % NBUF], out_hbm.at[pl.ds(base + ch * CH, CH)],
            osems[ch % NBUF])

    def compute(ch):
        buf = bufs[ch % NBUF]
        for jb in range(NJB):
            e0s = [emb_v[pl.ds((jb * JB + k) * LANES, LANES)]
                   for k in range(JB)]
            e1s = [emb_v[pl.ds(D + (jb * JB + k) * LANES, LANES)]
                   for k in range(JB)]

            @plsc.parallel_loop(0, CH, unroll=2)
            def tok(t):
                idxv = jnp.full((LANES,), ch * CH + t, jnp.int32)
                rowv = plsc.load_gather(tid_v, [idxv])
                mask = rowv != 0
                for k in range(JB):
                    sl = pl.ds((jb * JB + k) * LANES, LANES)
                    plsc.addupdate(buf.at[t, sl],
                                   jnp.where(mask, e1s[k], e0s[k]))

    start_in(0)
    start_in(1)
    emb_cp.wait()
    tid_cp.wait()
    for ch in range(NCHUNK):
        in_copies[ch].wait()
        compute(ch)
        start_out(ch)
        if ch + 2 < NCHUNK:
            if ch >= 1:
                out_copies[ch - 1].wait()
            start_in(ch + 2)
    for ch in range(NCHUNK - 3, NCHUNK):
        out_copies[ch].wait()


def kernel(x, type_ids, emb):
    x2 = x.reshape(NTOK, D)
    tid = type_ids.reshape(NTOK).astype(jnp.int32)
    mesh = plsc.VectorSubcoreMesh(core_axis_name="c", subcore_axis_name="s")
    out = pl.kernel(
        _sc_body,
        out_type=jax.ShapeDtypeStruct((NTOK, D), jnp.float32),
        mesh=mesh,
        compiler_params=pltpu.CompilerParams(needs_layout_passes=False),
        scratch_types=[
            pltpu.VMEM((2 * D,), jnp.float32),
            pltpu.VMEM((TPW,), jnp.int32),
            pltpu.VMEM((CH, D), jnp.float32),
            pltpu.VMEM((CH, D), jnp.float32),
            pltpu.VMEM((CH, D), jnp.float32),
            pltpu.SemaphoreType.DMA,
            pltpu.SemaphoreType.DMA,
            pltpu.SemaphoreType.DMA,
            pltpu.SemaphoreType.DMA,
            pltpu.SemaphoreType.DMA,
            pltpu.SemaphoreType.DMA,
        ],
    )(x2, tid, emb.reshape(2 * D))
    return out.reshape(B, L, D)


# speedup vs baseline: 1.0014x; 1.0014x over previous
import jax
import jax.numpy as jnp
from jax import lax
from jax.experimental import pallas as pl
from jax.experimental.pallas import tpu as pltpu
from jax.experimental.pallas import tpu_sc as plsc

B, L, D = 4, 4096, 1024
NTOK = B * L
LANES = 16
NC, NS = 2, 16
NW = NC * NS
TPW = NTOK // NW
CH = 32
NCHUNK = TPW // CH
NBUF = 3
JB = 16
NJB = (D // LANES) // JB


def _sc_body(x_hbm, tid_hbm, emb_hbm, out_hbm,
             emb_v, tid_v, buf0, buf1, buf2,
             si0, si1, si2, so0, so1, so2):
    c = lax.axis_index("c")
    s = lax.axis_index("s")
    wid = s * NC + c
    base = wid * TPW

    emb_cp = pltpu.async_copy(emb_hbm, emb_v, so0)
    tid_cp = pltpu.async_copy(tid_hbm.at[pl.ds(base, TPW)], tid_v, so1)

    bufs = (buf0, buf1, buf2)
    isems = (si0, si1, si2)
    osems = (so0, so1, so2)
    in_copies = [None] * NCHUNK
    out_copies = [None] * NCHUNK

    def start_in(ch):
        in_copies[ch] = pltpu.async_copy(
            x_hbm.at[pl.ds(base + ch * CH, CH)], bufs[ch % NBUF],
            isems[ch % NBUF])

    def start_out(ch):
        out_copies[ch] = pltpu.async_copy(
            bufs[ch % NBUF], out_hbm.at[pl.ds(base + ch * CH, CH)],
            osems[ch % NBUF])

    def compute(ch):
        buf = bufs[ch % NBUF]
        for jb in range(NJB):
            e0s = [emb_v[pl.ds((jb * JB + k) * LANES, LANES)]
                   for k in range(JB)]
            e1s = [emb_v[pl.ds(D + (jb * JB + k) * LANES, LANES)]
                   for k in range(JB)]

            @plsc.parallel_loop(0, CH, unroll=2)
            def tok(t):
                idxv = jnp.full((LANES,), ch * CH + t, jnp.int32)
                rowv = plsc.load_gather(tid_v, [idxv])
                mask = rowv != 0
                for k in range(JB):
                    sl = pl.ds((jb * JB + k) * LANES, LANES)
                    plsc.addupdate(buf.at[t, sl],
                                   jnp.where(mask, e1s[k], e0s[k]))

    start_in(0)
    start_in(1)
    emb_cp.wait()
    tid_cp.wait()
    for ch in range(NCHUNK):
        in_copies[ch].wait()
        compute(ch)
        start_out(ch)
        if ch + 2 < NCHUNK:
            if ch >= 1:
                out_copies[ch - 1].wait()
            start_in(ch + 2)
    for ch in range(NCHUNK - 3, NCHUNK):
        out_copies[ch].wait()


def kernel(x, type_ids, emb):
    x2 = x.reshape(NTOK, D)
    tid = type_ids.reshape(NTOK).astype(jnp.int32)
    mesh = plsc.VectorSubcoreMesh(core_axis_name="c", subcore_axis_name="s")
    out = pl.kernel(
        _sc_body,
        out_type=jax.ShapeDtypeStruct((NTOK, D), jnp.float32),
        mesh=mesh,
        compiler_params=pltpu.CompilerParams(
            needs_layout_passes=False, skip_device_barrier=True),
        scratch_types=[
            pltpu.VMEM((2 * D,), jnp.float32),
            pltpu.VMEM((TPW,), jnp.int32),
            pltpu.VMEM((CH, D), jnp.float32),
            pltpu.VMEM((CH, D), jnp.float32),
            pltpu.VMEM((CH, D), jnp.float32),
            pltpu.SemaphoreType.DMA,
            pltpu.SemaphoreType.DMA,
            pltpu.SemaphoreType.DMA,
            pltpu.SemaphoreType.DMA,
            pltpu.SemaphoreType.DMA,
            pltpu.SemaphoreType.DMA,
        ],
    )(x2, tid, emb.reshape(2 * D))
    return out.reshape(B, L, D)
